# baseline (speedup 1.0000x reference)
import jax
import jax.numpy as jnp
from jax.experimental import pallas as pl
from jax.experimental.pallas import tpu as pltpu

_NB = 16


def _lsh_vote_kernel(x_ref, w_ref, out_ref):
    x = x_ref[...]
    v_count = w_ref.shape[0]
    w = jnp.concatenate([w_ref[i] for i in range(v_count)], axis=-1)
    proj = jnp.dot(x, w, preferred_element_type=jnp.float32)
    bits = (proj > 0).astype(jnp.bfloat16)
    c_total = proj.shape[1]
    c = jax.lax.broadcasted_iota(jnp.int32, (v_count, c_total), 1)
    v = jax.lax.broadcasted_iota(jnp.int32, (v_count, c_total), 0)
    pow2 = jnp.left_shift(jnp.int32(1), c % _NB).astype(jnp.float32)
    packT = jnp.where(c // _NB == v, pow2, 0.0).astype(jnp.bfloat16)
    votes_t = jax.lax.dot_general(
        packT, bits, (((1,), (1,)), ((), ())),
        preferred_element_type=jnp.float32)
    out_ref[...] = votes_t.astype(jnp.int32)


def kernel(x, W):
    Q, D = x.shape
    V, _, B = W.shape
    BQ = 8192
    return pl.pallas_call(
        _lsh_vote_kernel,
        grid=(Q // BQ,),
        in_specs=[
            pl.BlockSpec((BQ, D), lambda i: (i, 0)),
            pl.BlockSpec((V, D, B), lambda i: (0, 0, 0)),
        ],
        out_specs=pl.BlockSpec((V, BQ), lambda i: (0, i)),
        out_shape=jax.ShapeDtypeStruct((V, Q), jnp.int32),
        compiler_params=pltpu.CompilerParams(
            dimension_semantics=("parallel",)),
    )(x, W)


# speedup vs baseline: 1.1220x; 1.1220x over previous
import jax
import jax.numpy as jnp
from jax.experimental import pallas as pl
from jax.experimental.pallas import tpu as pltpu

_NB = 16


def _lsh_vote_kernel(x_ref, w_ref, out_ref):
    x = x_ref[...]
    w = w_ref[...]
    proj = jnp.dot(x, w, preferred_element_type=jnp.float32)
    bits = (proj > 0).astype(jnp.bfloat16)
    c_total = w.shape[1]
    v_total = c_total // _NB
    c = jax.lax.broadcasted_iota(jnp.int32, (v_total, c_total), 1)
    v = jax.lax.broadcasted_iota(jnp.int32, (v_total, c_total), 0)
    pow2 = jnp.left_shift(jnp.int32(1), c % _NB).astype(jnp.float32)
    packT = jnp.where(c // _NB == v, pow2, 0.0).astype(jnp.bfloat16)
    votes_t = jax.lax.dot_general(
        packT, bits, (((1,), (1,)), ((), ())),
        preferred_element_type=jnp.float32)
    out_ref[...] = votes_t.astype(jnp.int32)


def kernel(x, W):
    Q, D = x.shape
    V, _, B = W.shape
    W2 = jnp.transpose(W, (1, 0, 2)).reshape(D, V * B)
    BQ = 8192
    return pl.pallas_call(
        _lsh_vote_kernel,
        grid=(Q // BQ,),
        in_specs=[
            pl.BlockSpec((BQ, D), lambda i: (i, 0)),
            pl.BlockSpec((D, V * B), lambda i: (0, 0)),
        ],
        out_specs=pl.BlockSpec((V, BQ), lambda i: (0, i)),
        out_shape=jax.ShapeDtypeStruct((V, Q), jnp.int32),
        compiler_params=pltpu.CompilerParams(
            dimension_semantics=("parallel",)),
    )(x, W2)


# speedup vs baseline: 1.1267x; 1.0042x over previous
import jax
import jax.numpy as jnp
from jax.experimental import pallas as pl
from jax.experimental.pallas import tpu as pltpu

_NB = 16


_NSTREAM = 4


def _lsh_vote_kernel(*refs):
    x_refs = refs[:_NSTREAM]
    w_ref, out_ref = refs[_NSTREAM], refs[_NSTREAM + 1]
    w = w_ref[...]
    c_total = w.shape[1]
    v_total = c_total // _NB
    c = jax.lax.broadcasted_iota(jnp.int32, (v_total, c_total), 1)
    v = jax.lax.broadcasted_iota(jnp.int32, (v_total, c_total), 0)
    pow2 = jnp.left_shift(jnp.int32(1), c % _NB).astype(jnp.float32)
    packT = jnp.where(c // _NB == v, pow2, 0.0).astype(jnp.bfloat16)
    for j, x_ref in enumerate(x_refs):
        x = x_ref[...]
        cq = x.shape[0]
        proj = jnp.dot(x, w, preferred_element_type=jnp.float32)
        bits = (proj > 0).astype(jnp.bfloat16)
        votes_t = jax.lax.dot_general(
            packT, bits, (((1,), (1,)), ((), ())),
            preferred_element_type=jnp.float32)
        out_ref[:, j * cq:(j + 1) * cq] = votes_t.astype(jnp.int32)


def kernel(x, W):
    Q, D = x.shape
    V, _, B = W.shape
    W2 = jnp.transpose(W, (1, 0, 2)).reshape(D, V * B)
    BQ = 8192
    CQ = BQ // _NSTREAM
    x_specs = [
        pl.BlockSpec((CQ, D), lambda i, j=j: (_NSTREAM * i + j, 0))
        for j in range(_NSTREAM)
    ]
    return pl.pallas_call(
        _lsh_vote_kernel,
        grid=(Q // BQ,),
        in_specs=x_specs + [pl.BlockSpec((D, V * B), lambda i: (0, 0))],
        out_specs=pl.BlockSpec((V, BQ), lambda i: (0, i)),
        out_shape=jax.ShapeDtypeStruct((V, Q), jnp.int32),
        compiler_params=pltpu.CompilerParams(
            dimension_semantics=("parallel",)),
    )(*([x] * _NSTREAM), W2)
